# Initial kernel scaffold; baseline (speedup 1.0000x reference)
import jax
import jax.numpy as jnp
from jax.experimental import pallas as pl


def kernel(feat, edge_index, wl, bl, wr, br):
    raise NotImplementedError("write your pallas kernel here")


# speedup vs baseline: 13.8208x; 13.8208x over previous
import functools

import jax
import jax.numpy as jnp
from jax import lax
from jax.experimental import pallas as pl
from jax.experimental.pallas import tpu as pltpu
from jax.experimental.pallas import tpu_sc as plsc

N = 10000
E = 320000
D = 128
L = 16
NC = 2
NS = 16
NW = NC * NS
EPW = E // NW
C = 80
NCHUNK = EPW // C
N_PAD = 10240
RPT = N_PAD // NS
RCH = 128
NRC = RPT // RCH


def _elr_body(feat_ref, w_ref, b_ref, out_ref):
    out_ref[...] = (
        jnp.dot(feat_ref[...], w_ref[...], preferred_element_type=jnp.float32)
        + b_ref[...]
    )


def _combine_body(p_ref, o_ref):
    o_ref[...] = p_ref[0, :N] + p_ref[1, :N]


_sc_mesh = plsc.VectorSubcoreMesh(
    core_axis_name="c", subcore_axis_name="s", num_cores=NC, num_subcores=NS
)


@functools.partial(
    pl.kernel,
    out_type=jax.ShapeDtypeStruct((NC * N_PAD, D), jnp.float32),
    mesh=_sc_mesh,
    scratch_types=[
        pltpu.VMEM((N,), jnp.float32),
        pltpu.VMEM((N,), jnp.float32),
        pltpu.VMEM((C,), jnp.int32),
        pltpu.VMEM((C,), jnp.int32),
        pltpu.VMEM((C,), jnp.float32),
        pltpu.VMEM((C, D), jnp.float32),
        pltpu.VMEM((RCH, D), jnp.float32),
        pltpu.VMEM_SHARED((N_PAD, D), jnp.float32),
        pltpu.SemaphoreType.DMA,
    ],
    compiler_params=pltpu.CompilerParams(needs_layout_passes=False),
)
def _sc_edges(src_hbm, dst_hbm, el_hbm, er_hbm, feat_hbm, out_hbm,
              el_v, er_v, src_v, dst_v, a_v, rows_v, st_v, acc, sem):
    cid = lax.axis_index("c")
    sid = lax.axis_index("s")
    wid = sid * NC + cid

    def _zrow(r, carry):
        for g in range(D // L):
            st_v[r, pl.ds(g * L, L)] = jnp.zeros((L,), jnp.float32)
        return carry

    lax.fori_loop(0, RCH, _zrow, 0)
    row0 = sid * RPT
    for j in range(NRC):
        pltpu.sync_copy(st_v, acc.at[pl.ds(row0 + j * RCH, RCH)])

    pltpu.sync_copy(el_hbm, el_v)
    pltpu.sync_copy(er_hbm, er_v)
    plsc.subcore_barrier()

    ebase = wid * EPW

    def _chunk(i, carry):
        base = ebase + i * C
        pltpu.sync_copy(src_hbm.at[pl.ds(base, C)], src_v)
        pltpu.sync_copy(dst_hbm.at[pl.ds(base, C)], dst_v)
        pltpu.async_copy(feat_hbm.at[src_v], rows_v, sem).wait()
        for g in range(C // L):
            sv = src_v[pl.ds(g * L, L)]
            dv = dst_v[pl.ds(g * L, L)]
            e = plsc.load_gather(el_v, [sv]) + plsc.load_gather(er_v, [dv])
            a_v[pl.ds(g * L, L)] = jnp.where(e > 0, e, 0.2 * e)

        def _row(r, rcarry):
            av = plsc.load_gather(a_v, [jnp.zeros((L,), jnp.int32) + r])
            for g in range(D // L):
                sl = (r, pl.ds(g * L, L))
                rows_v[sl] = rows_v[sl] * av
            return rcarry

        lax.fori_loop(0, C, _row, 0)
        pltpu.sync_copy(rows_v, acc.at[dst_v], add=True)
        return carry

    lax.fori_loop(0, NCHUNK, _chunk, 0)

    plsc.subcore_barrier()
    for j in range(NRC):
        r0 = sid * RPT + j * RCH
        pltpu.sync_copy(acc.at[pl.ds(r0, RCH)], st_v)
        pltpu.sync_copy(st_v, out_hbm.at[pl.ds(cid * N_PAD + r0, RCH)])


def kernel(feat, edge_index, wl, bl, wr, br):
    w2 = jnp.concatenate([wl, wr], axis=1)
    b2 = jnp.concatenate([bl, br]).reshape(1, 2)
    elr = pl.pallas_call(
        _elr_body,
        out_shape=jax.ShapeDtypeStruct((N, 2), jnp.float32),
    )(feat, w2, b2)
    el = elr[:, 0]
    er = elr[:, 1]
    src = edge_index[0].astype(jnp.int32)
    dst = edge_index[1].astype(jnp.int32)
    parts = _sc_edges(src, dst, el, er, feat)
    out = pl.pallas_call(
        _combine_body,
        out_shape=jax.ShapeDtypeStruct((N, D), jnp.float32),
    )(parts.reshape(NC, N_PAD, D))
    return out


# speedup vs baseline: 22.2675x; 1.6112x over previous
import functools

import jax
import jax.numpy as jnp
from jax import lax
from jax.experimental import pallas as pl
from jax.experimental.pallas import tpu as pltpu
from jax.experimental.pallas import tpu_sc as plsc

N = 10000
E = 320000
D = 128
L = 16
NC = 2
NS = 16
NW = NC * NS
EPW = E // NW
C = 80
NCHUNK = EPW // C
N_PAD = 10240
RPT = N_PAD // NS
RCH = C
NRC = RPT // RCH


def _elr_body(feat_ref, w_ref, b_ref, out_ref):
    out_ref[...] = (
        jnp.dot(feat_ref[...], w_ref[...], preferred_element_type=jnp.float32)
        + b_ref[...]
    )


def _combine_body(p_ref, o_ref):
    o_ref[...] = p_ref[0, :N] + p_ref[1, :N]


_sc_mesh = plsc.VectorSubcoreMesh(
    core_axis_name="c", subcore_axis_name="s", num_cores=NC, num_subcores=NS
)


@functools.partial(
    pl.kernel,
    out_type=jax.ShapeDtypeStruct((NC * N_PAD, D), jnp.float32),
    mesh=_sc_mesh,
    scratch_types=[
        pltpu.VMEM((N,), jnp.float32),
        pltpu.VMEM((N,), jnp.float32),
        pltpu.VMEM((C,), jnp.int32),
        pltpu.VMEM((C,), jnp.int32),
        pltpu.VMEM((C,), jnp.float32),
        pltpu.VMEM((C, D), jnp.float32),
        pltpu.VMEM((C,), jnp.int32),
        pltpu.VMEM((C,), jnp.int32),
        pltpu.VMEM((C,), jnp.float32),
        pltpu.VMEM((C, D), jnp.float32),
        pltpu.VMEM_SHARED((N_PAD, D), jnp.float32),
        pltpu.SemaphoreType.DMA,
        pltpu.SemaphoreType.DMA,
    ],
    compiler_params=pltpu.CompilerParams(needs_layout_passes=False),
)
def _sc_edges(src_hbm, dst_hbm, el_hbm, er_hbm, feat_hbm, out_hbm,
              el_v, er_v, src0, dst0, a0, rows0, src1, dst1, a1, rows1,
              acc, sem0, sem1):
    cid = lax.axis_index("c")
    sid = lax.axis_index("s")
    wid = sid * NC + cid
    bufs = ((src0, dst0, a0, rows0, sem0), (src1, dst1, a1, rows1, sem1))

    def _zrow(r, carry):
        for g in range(D // L):
            rows0[r, pl.ds(g * L, L)] = jnp.zeros((L,), jnp.float32)
        return carry

    lax.fori_loop(0, RCH, _zrow, 0)
    row0 = sid * RPT
    for j in range(NRC):
        pltpu.sync_copy(rows0, acc.at[pl.ds(row0 + j * RCH, RCH)])

    pltpu.sync_copy(el_hbm, el_v)
    pltpu.sync_copy(er_hbm, er_v)
    plsc.subcore_barrier()

    ebase = wid * EPW

    for b in range(2):
        sv, dv, av, rv, sm = bufs[b]
        base = ebase + b * C
        pltpu.sync_copy(src_hbm.at[pl.ds(base, C)], sv)
        pltpu.sync_copy(dst_hbm.at[pl.ds(base, C)], dv)
        pltpu.async_copy(feat_hbm.at[sv], rv, sm)

    def _outer(k, carry):
        for b in range(2):
            i = 2 * k + b
            sv, dv, av, rv, sm = bufs[b]

            @pl.when(i < NCHUNK)
            def _do():
                pltpu.make_async_copy(feat_hbm.at[sv], rv, sm).wait()
                for g in range(C // L):
                    s16 = sv[pl.ds(g * L, L)]
                    d16 = dv[pl.ds(g * L, L)]
                    e = (plsc.load_gather(el_v, [s16])
                         + plsc.load_gather(er_v, [d16]))
                    av[pl.ds(g * L, L)] = jnp.where(e > 0, e, 0.2 * e)

                @plsc.parallel_loop(0, C, unroll=4)
                def _row(r):
                    bc = plsc.load_gather(av, [jnp.zeros((L,), jnp.int32) + r])
                    for g in range(D // L):
                        sl = (r, pl.ds(g * L, L))
                        rv[sl] = rv[sl] * bc

                pltpu.sync_copy(rv, acc.at[dv], add=True)

                @pl.when(i + 2 < NCHUNK)
                def _pf():
                    base2 = ebase + (i + 2) * C
                    pltpu.sync_copy(src_hbm.at[pl.ds(base2, C)], sv)
                    pltpu.sync_copy(dst_hbm.at[pl.ds(base2, C)], dv)
                    pltpu.async_copy(feat_hbm.at[sv], rv, sm)

        return carry

    lax.fori_loop(0, (NCHUNK + 1) // 2, _outer, 0)

    plsc.subcore_barrier()
    for j in range(NRC):
        r0 = sid * RPT + j * RCH
        pltpu.sync_copy(acc.at[pl.ds(r0, RCH)], rows0)
        pltpu.sync_copy(rows0, out_hbm.at[pl.ds(cid * N_PAD + r0, RCH)])


def kernel(feat, edge_index, wl, bl, wr, br):
    w2 = jnp.concatenate([wl, wr], axis=1)
    b2 = jnp.concatenate([bl, br]).reshape(1, 2)
    elr = pl.pallas_call(
        _elr_body,
        out_shape=jax.ShapeDtypeStruct((N, 2), jnp.float32),
    )(feat, w2, b2)
    el = elr[:, 0]
    er = elr[:, 1]
    src = edge_index[0].astype(jnp.int32)
    dst = edge_index[1].astype(jnp.int32)
    parts = _sc_edges(src, dst, el, er, feat)
    out = pl.pallas_call(
        _combine_body,
        out_shape=jax.ShapeDtypeStruct((N, D), jnp.float32),
    )(parts.reshape(NC, N_PAD, D))
    return out


# speedup vs baseline: 26.5131x; 1.1907x over previous
import functools

import jax
import jax.numpy as jnp
from jax import lax
from jax.experimental import pallas as pl
from jax.experimental.pallas import tpu as pltpu
from jax.experimental.pallas import tpu_sc as plsc

N = 10000
E = 320000
D = 128
L = 16
NC = 2
NS = 16
NW = NC * NS
EPW = E // NW
C = 80
NCHUNK = EPW // C
N_PAD = 10240
RPT = N_PAD // NS
RCH = C
NRC = RPT // RCH
NBUF = 4


def _elr_body(feat_ref, w_ref, b_ref, out_ref):
    out_ref[...] = (
        jnp.dot(feat_ref[...], w_ref[...], preferred_element_type=jnp.float32)
        + b_ref[...]
    )


def _combine_body(p_ref, o_ref):
    o_ref[...] = p_ref[0, :N] + p_ref[1, :N]


_sc_mesh = plsc.VectorSubcoreMesh(
    core_axis_name="c", subcore_axis_name="s", num_cores=NC, num_subcores=NS
)


@functools.partial(
    pl.kernel,
    out_type=jax.ShapeDtypeStruct((NC * N_PAD, D), jnp.float32),
    mesh=_sc_mesh,
    scratch_types=[
        [
            dict(
                src=pltpu.VMEM((C,), jnp.int32),
                dst=pltpu.VMEM((C,), jnp.int32),
                elg=pltpu.VMEM((C,), jnp.float32),
                erg=pltpu.VMEM((C,), jnp.float32),
                a=pltpu.VMEM((C,), jnp.float32),
                rows=pltpu.VMEM((C, D), jnp.float32),
                gsem=pltpu.SemaphoreType.DMA,
                ssem=pltpu.SemaphoreType.DMA,
            )
            for _ in range(NBUF)
        ],
        pltpu.VMEM_SHARED((N_PAD, D), jnp.float32),
    ],
    compiler_params=pltpu.CompilerParams(needs_layout_passes=False),
)
def _sc_edges(src_hbm, dst_hbm, el_hbm, er_hbm, feat_hbm, out_hbm, bufs, acc):
    cid = lax.axis_index("c")
    sid = lax.axis_index("s")
    wid = sid * NC + cid
    ebase = wid * EPW

    def _fetch(buf, chunk):
        base = ebase + chunk * C
        pltpu.sync_copy(src_hbm.at[pl.ds(base, C)], buf["src"])
        pltpu.sync_copy(dst_hbm.at[pl.ds(base, C)], buf["dst"])
        pltpu.async_copy(el_hbm.at[buf["src"]], buf["elg"], buf["gsem"])
        pltpu.async_copy(er_hbm.at[buf["dst"]], buf["erg"], buf["gsem"])
        pltpu.async_copy(feat_hbm.at[buf["src"]], buf["rows"], buf["gsem"])

    def _fetch_wait(buf):
        pltpu.make_async_copy(el_hbm.at[buf["src"]], buf["elg"], buf["gsem"]).wait()
        pltpu.make_async_copy(er_hbm.at[buf["dst"]], buf["erg"], buf["gsem"]).wait()
        pltpu.make_async_copy(feat_hbm.at[buf["src"]], buf["rows"], buf["gsem"]).wait()

    def _scatter_wait(buf):
        pltpu.make_async_copy(buf["rows"], acc.at[buf["dst"]], buf["ssem"]).wait()

    z = bufs[0]["rows"]

    def _zrow(r, carry):
        for g in range(D // L):
            z[r, pl.ds(g * L, L)] = jnp.zeros((L,), jnp.float32)
        return carry

    lax.fori_loop(0, RCH, _zrow, 0)
    row0 = sid * RPT
    for j in range(NRC):
        pltpu.sync_copy(z, acc.at[pl.ds(row0 + j * RCH, RCH)])
    plsc.subcore_barrier()

    for b in range(2):
        _fetch(bufs[b], b)

    def _slot(i, carry):
        for b in range(NBUF):

            @pl.when(i % NBUF == b)
            def _do():
                buf = bufs[b]
                pbuf = bufs[(b + 2) % NBUF]

                @pl.when(i + 2 < NCHUNK)
                def _pf():
                    @pl.when(i >= 2)
                    def _dr():
                        _scatter_wait(pbuf)

                    _fetch(pbuf, i + 2)

                _fetch_wait(buf)
                for g in range(C // L):
                    e = (buf["elg"][pl.ds(g * L, L)]
                         + buf["erg"][pl.ds(g * L, L)])
                    buf["a"][pl.ds(g * L, L)] = jnp.where(e > 0, e, 0.2 * e)

                av = buf["a"]
                rv = buf["rows"]

                @plsc.parallel_loop(0, C, unroll=4)
                def _row(r):
                    bc = plsc.load_gather(av, [jnp.zeros((L,), jnp.int32) + r])
                    for g in range(D // L):
                        sl = (r, pl.ds(g * L, L))
                        rv[sl] = rv[sl] * bc

                pltpu.async_copy(rv, acc.at[buf["dst"]], buf["ssem"], add=True)

        return carry

    lax.fori_loop(0, NCHUNK, _slot, 0)
    for b in range(NBUF):
        _scatter_wait(bufs[b])

    plsc.subcore_barrier()
    st = bufs[0]["rows"]
    for j in range(NRC):
        r0 = sid * RPT + j * RCH
        pltpu.sync_copy(acc.at[pl.ds(r0, RCH)], st)
        pltpu.sync_copy(st, out_hbm.at[pl.ds(cid * N_PAD + r0, RCH)])


def kernel(feat, edge_index, wl, bl, wr, br):
    w2 = jnp.concatenate([wl, wr], axis=1)
    b2 = jnp.concatenate([bl, br]).reshape(1, 2)
    elr = pl.pallas_call(
        _elr_body,
        out_shape=jax.ShapeDtypeStruct((N, 2), jnp.float32),
    )(feat, w2, b2)
    el = elr[:, 0]
    er = elr[:, 1]
    src = edge_index[0].astype(jnp.int32)
    dst = edge_index[1].astype(jnp.int32)
    parts = _sc_edges(src, dst, el, er, feat)
    out = pl.pallas_call(
        _combine_body,
        out_shape=jax.ShapeDtypeStruct((N, D), jnp.float32),
    )(parts.reshape(NC, N_PAD, D))
    return out


# speedup vs baseline: 36.2378x; 1.3668x over previous
import functools

import jax
import jax.numpy as jnp
from jax import lax
from jax.experimental import pallas as pl
from jax.experimental.pallas import tpu as pltpu
from jax.experimental.pallas import tpu_sc as plsc

N = 10000
E = 320000
D = 128
L = 16
NC = 2
NS = 16
NW = NC * NS
EPW = E // NW
C = 80
NCHUNK = EPW // C
N_PAD = 10240
RPT = N_PAD // NS
RCH = C
NRC = RPT // RCH
NBUF = 4
NIB = 8


def _elr_body(feat_ref, w_ref, b_ref, out_ref):
    out_ref[...] = (
        jnp.dot(feat_ref[...], w_ref[...], preferred_element_type=jnp.float32)
        + b_ref[...]
    )


def _combine_body(p_ref, o_ref):
    o_ref[...] = p_ref[0, :N] + p_ref[1, :N]


_sc_mesh = plsc.VectorSubcoreMesh(
    core_axis_name="c", subcore_axis_name="s", num_cores=NC, num_subcores=NS
)


@functools.partial(
    pl.kernel,
    out_type=jax.ShapeDtypeStruct((NC * N_PAD, D), jnp.float32),
    mesh=_sc_mesh,
    scratch_types=[
        [
            dict(
                elg=pltpu.VMEM((C,), jnp.float32),
                erg=pltpu.VMEM((C,), jnp.float32),
                a=pltpu.VMEM((C,), jnp.float32),
                rows=pltpu.VMEM((C, D), jnp.float32),
                gsem=pltpu.SemaphoreType.DMA,
                ssem=pltpu.SemaphoreType.DMA,
            )
            for _ in range(NBUF)
        ],
        [
            dict(
                src=pltpu.VMEM((C,), jnp.int32),
                dst=pltpu.VMEM((C,), jnp.int32),
                isem=pltpu.SemaphoreType.DMA,
            )
            for _ in range(NIB)
        ],
        pltpu.VMEM_SHARED((N_PAD, D), jnp.float32),
    ],
    compiler_params=pltpu.CompilerParams(needs_layout_passes=False),
)
def _sc_edges(src_hbm, dst_hbm, el_hbm, er_hbm, feat_hbm, out_hbm,
              rbufs, ibufs, acc):
    cid = lax.axis_index("c")
    sid = lax.axis_index("s")
    wid = sid * NC + cid
    ebase = wid * EPW

    def _fire_idx(ib, chunk):
        base = ebase + chunk * C
        pltpu.async_copy(src_hbm.at[pl.ds(base, C)], ib["src"], ib["isem"])
        pltpu.async_copy(dst_hbm.at[pl.ds(base, C)], ib["dst"], ib["isem"])

    def _wait_idx(ib):
        pltpu.make_async_copy(src_hbm.at[pl.ds(0, C)], ib["src"], ib["isem"]).wait()
        pltpu.make_async_copy(dst_hbm.at[pl.ds(0, C)], ib["dst"], ib["isem"]).wait()

    def _fire_gather(rb, ib):
        pltpu.async_copy(el_hbm.at[ib["src"]], rb["elg"], rb["gsem"])
        pltpu.async_copy(er_hbm.at[ib["dst"]], rb["erg"], rb["gsem"])
        pltpu.async_copy(feat_hbm.at[ib["src"]], rb["rows"], rb["gsem"])

    def _wait_gather(rb):
        pltpu.make_async_copy(el_hbm.at[pl.ds(0, C)], rb["elg"], rb["gsem"]).wait()
        pltpu.make_async_copy(er_hbm.at[pl.ds(0, C)], rb["erg"], rb["gsem"]).wait()
        pltpu.make_async_copy(feat_hbm.at[pl.ds(0, C)], rb["rows"], rb["gsem"]).wait()

    def _wait_scatter(rb):
        pltpu.make_async_copy(feat_hbm.at[pl.ds(0, C)], rb["rows"], rb["ssem"]).wait()

    z = rbufs[0]["rows"]

    def _zrow(r, carry):
        for g in range(D // L):
            z[r, pl.ds(g * L, L)] = jnp.zeros((L,), jnp.float32)
        return carry

    lax.fori_loop(0, RCH, _zrow, 0)
    row0 = sid * RPT
    for j in range(NRC):
        pltpu.sync_copy(z, acc.at[pl.ds(row0 + j * RCH, RCH)])
    plsc.subcore_barrier()

    for j in range(4):
        _fire_idx(ibufs[j], j)
    for j in range(2):
        _wait_idx(ibufs[j])
        _fire_gather(rbufs[j], ibufs[j])

    def _slot(i, carry):
        m8 = lax.rem(i, jnp.int32(NIB))
        for s in range(NIB):
            b = s % NBUF

            @pl.when(m8 == s)
            def _case():
                rb = rbufs[b]
                ib = ibufs[s]

                @pl.when(i + 4 < NCHUNK)
                def _pf_idx():
                    _fire_idx(ibufs[(s + 4) % NIB], i + 4)

                @pl.when(i + 2 < NCHUNK)
                def _pf_rows():
                    @pl.when(i >= 2)
                    def _dr():
                        _wait_scatter(rbufs[(b + 2) % NBUF])

                    ib2 = ibufs[(s + 2) % NIB]
                    _wait_idx(ib2)
                    _fire_gather(rbufs[(b + 2) % NBUF], ib2)

                _wait_gather(rb)
                for g in range(C // L):
                    e = (rb["elg"][pl.ds(g * L, L)]
                         + rb["erg"][pl.ds(g * L, L)])
                    rb["a"][pl.ds(g * L, L)] = jnp.where(e > 0, e, 0.2 * e)

                av = rb["a"]
                rv = rb["rows"]

                @plsc.parallel_loop(0, C, unroll=4)
                def _row(r):
                    bc = plsc.load_gather(av, [jnp.zeros((L,), jnp.int32) + r])
                    for g in range(D // L):
                        sl = (r, pl.ds(g * L, L))
                        rv[sl] = rv[sl] * bc

                pltpu.async_copy(rv, acc.at[ib["dst"]], rb["ssem"], add=True)

        return carry

    lax.fori_loop(0, NCHUNK, _slot, 0)
    for b in range(NBUF):
        _wait_scatter(rbufs[b])

    plsc.subcore_barrier()
    st = rbufs[0]["rows"]
    for j in range(NRC):
        r0 = sid * RPT + j * RCH
        pltpu.sync_copy(acc.at[pl.ds(r0, RCH)], st)
        pltpu.sync_copy(st, out_hbm.at[pl.ds(cid * N_PAD + r0, RCH)])


def kernel(feat, edge_index, wl, bl, wr, br):
    w2 = jnp.concatenate([wl, wr], axis=1)
    b2 = jnp.concatenate([bl, br]).reshape(1, 2)
    elr = pl.pallas_call(
        _elr_body,
        out_shape=jax.ShapeDtypeStruct((N, 2), jnp.float32),
    )(feat, w2, b2)
    el = elr[:, 0]
    er = elr[:, 1]
    src = edge_index[0].astype(jnp.int32)
    dst = edge_index[1].astype(jnp.int32)
    parts = _sc_edges(src, dst, el, er, feat)
    out = pl.pallas_call(
        _combine_body,
        out_shape=jax.ShapeDtypeStruct((N, D), jnp.float32),
    )(parts.reshape(NC, N_PAD, D))
    return out


# speedup vs baseline: 39.0317x; 1.0771x over previous
import functools

import jax
import jax.numpy as jnp
from jax import lax
from jax.experimental import pallas as pl
from jax.experimental.pallas import tpu as pltpu
from jax.experimental.pallas import tpu_sc as plsc

N = 10000
E = 320000
D = 128
L = 16
NC = 2
NS = 16
NW = NC * NS
EPW = E // NW
C = 80
NCHUNK = EPW // C
N_PAD = 10240
RPT = N_PAD // NS
RCH = C
NRC = RPT // RCH
NBUF = 4
NIB = 8


def _elr_body(feat_ref, w_ref, b_ref, el_ref, er_ref):
    elr = (
        jnp.dot(feat_ref[...], w_ref[...], preferred_element_type=jnp.float32)
        + b_ref[...]
    )
    el_ref[...] = elr[:, 0]
    er_ref[...] = elr[:, 1]


def _combine_body(p_ref, o_ref):
    o_ref[...] = p_ref[0, :N] + p_ref[1, :N]


_sc_mesh = plsc.VectorSubcoreMesh(
    core_axis_name="c", subcore_axis_name="s", num_cores=NC, num_subcores=NS
)


@functools.partial(
    pl.kernel,
    out_type=jax.ShapeDtypeStruct((NC * N_PAD, D), jnp.float32),
    mesh=_sc_mesh,
    scratch_types=[
        [
            dict(
                elg=pltpu.VMEM((C,), jnp.float32),
                erg=pltpu.VMEM((C,), jnp.float32),
                a=pltpu.VMEM((C,), jnp.float32),
                rows=pltpu.VMEM((C, D), jnp.float32),
                gsem=pltpu.SemaphoreType.DMA,
                ssem=pltpu.SemaphoreType.DMA,
            )
            for _ in range(NBUF)
        ],
        [
            dict(
                src=pltpu.VMEM((C,), jnp.int32),
                dst=pltpu.VMEM((C,), jnp.int32),
                isem=pltpu.SemaphoreType.DMA,
            )
            for _ in range(NIB)
        ],
        pltpu.VMEM_SHARED((N_PAD, D), jnp.float32),
    ],
    compiler_params=pltpu.CompilerParams(needs_layout_passes=False),
)
def _sc_edges(ei_hbm, el_hbm, er_hbm, feat_hbm, out_hbm,
              rbufs, ibufs, acc):
    cid = lax.axis_index("c")
    sid = lax.axis_index("s")
    wid = sid * NC + cid
    ebase = wid * EPW

    def _fire_idx(ib, chunk):
        base = ebase + chunk * C
        pltpu.async_copy(ei_hbm.at[pl.ds(base, C)], ib["src"], ib["isem"])
        pltpu.async_copy(ei_hbm.at[pl.ds(E + base, C)], ib["dst"], ib["isem"])

    def _wait_idx(ib):
        pltpu.make_async_copy(ei_hbm.at[pl.ds(0, C)], ib["src"], ib["isem"]).wait()
        pltpu.make_async_copy(ei_hbm.at[pl.ds(0, C)], ib["dst"], ib["isem"]).wait()

    def _fire_gather(rb, ib):
        pltpu.async_copy(el_hbm.at[ib["src"]], rb["elg"], rb["gsem"])
        pltpu.async_copy(er_hbm.at[ib["dst"]], rb["erg"], rb["gsem"])
        pltpu.async_copy(feat_hbm.at[ib["src"]], rb["rows"], rb["gsem"])

    def _wait_gather(rb):
        pltpu.make_async_copy(el_hbm.at[pl.ds(0, C)], rb["elg"], rb["gsem"]).wait()
        pltpu.make_async_copy(er_hbm.at[pl.ds(0, C)], rb["erg"], rb["gsem"]).wait()
        pltpu.make_async_copy(feat_hbm.at[pl.ds(0, C)], rb["rows"], rb["gsem"]).wait()

    def _wait_scatter(rb):
        pltpu.make_async_copy(feat_hbm.at[pl.ds(0, C)], rb["rows"], rb["ssem"]).wait()

    z = rbufs[0]["rows"]

    def _zrow(r, carry):
        for g in range(D // L):
            z[r, pl.ds(g * L, L)] = jnp.zeros((L,), jnp.float32)
        return carry

    lax.fori_loop(0, RCH, _zrow, 0)
    row0 = sid * RPT
    for j in range(NRC):
        pltpu.sync_copy(z, acc.at[pl.ds(row0 + j * RCH, RCH)])
    plsc.subcore_barrier()

    for j in range(4):
        _fire_idx(ibufs[j], j)
    for j in range(2):
        _wait_idx(ibufs[j])
        _fire_gather(rbufs[j], ibufs[j])

    def _slot(i, carry):
        m8 = lax.rem(i, jnp.int32(NIB))
        for s in range(NIB):
            b = s % NBUF

            @pl.when(m8 == s)
            def _case():
                rb = rbufs[b]
                ib = ibufs[s]

                @pl.when(i + 4 < NCHUNK)
                def _pf_idx():
                    _fire_idx(ibufs[(s + 4) % NIB], i + 4)

                @pl.when(i + 2 < NCHUNK)
                def _pf_rows():
                    @pl.when(i >= 2)
                    def _dr():
                        _wait_scatter(rbufs[(b + 2) % NBUF])

                    ib2 = ibufs[(s + 2) % NIB]
                    _wait_idx(ib2)
                    _fire_gather(rbufs[(b + 2) % NBUF], ib2)

                _wait_gather(rb)
                for g in range(C // L):
                    e = (rb["elg"][pl.ds(g * L, L)]
                         + rb["erg"][pl.ds(g * L, L)])
                    rb["a"][pl.ds(g * L, L)] = jnp.where(e > 0, e, 0.2 * e)

                av = rb["a"]
                rv = rb["rows"]

                @plsc.parallel_loop(0, C, unroll=8)
                def _row(r):
                    bc = plsc.load_gather(av, [jnp.zeros((L,), jnp.int32) + r])
                    for g in range(D // L):
                        sl = (r, pl.ds(g * L, L))
                        rv[sl] = rv[sl] * bc

                pltpu.async_copy(rv, acc.at[ib["dst"]], rb["ssem"], add=True)

        return carry

    lax.fori_loop(0, NCHUNK, _slot, 0)
    for b in range(NBUF):
        _wait_scatter(rbufs[b])

    plsc.subcore_barrier()
    st = rbufs[0]["rows"]
    for j in range(NRC):
        r0 = sid * RPT + j * RCH
        pltpu.sync_copy(acc.at[pl.ds(r0, RCH)], st)
        pltpu.sync_copy(st, out_hbm.at[pl.ds(cid * N_PAD + r0, RCH)])


def kernel(feat, edge_index, wl, bl, wr, br):
    w2 = jnp.concatenate([wl, wr], axis=1)
    b2 = jnp.concatenate([bl, br]).reshape(1, 2)
    el, er = pl.pallas_call(
        _elr_body,
        out_shape=(
            jax.ShapeDtypeStruct((N,), jnp.float32),
            jax.ShapeDtypeStruct((N,), jnp.float32),
        ),
    )(feat, w2, b2)
    ei = edge_index.astype(jnp.int32).reshape(2 * E)
    parts = _sc_edges(ei, el, er, feat)
    out = pl.pallas_call(
        _combine_body,
        out_shape=jax.ShapeDtypeStruct((N, D), jnp.float32),
    )(parts.reshape(NC, N_PAD, D))
    return out


# speedup vs baseline: 39.0432x; 1.0003x over previous
import functools

import jax
import jax.numpy as jnp
from jax import lax
from jax.experimental import pallas as pl
from jax.experimental.pallas import tpu as pltpu
from jax.experimental.pallas import tpu_sc as plsc

N = 10000
E = 320000
D = 128
L = 16
NC = 2
NS = 16
NW = NC * NS
EPW = E // NW
C = 80
NCHUNK = EPW // C
N_PAD = 10240
RPT = N_PAD // NS
RCH = C
NRC = RPT // RCH
NBUF = 4
NIB = 8


def _elr_body(feat_ref, w_ref, b_ref, el_ref, er_ref):
    elr = (
        jnp.dot(feat_ref[...], w_ref[...], preferred_element_type=jnp.float32)
        + b_ref[...]
    )
    el_ref[...] = elr[:, 0]
    er_ref[...] = elr[:, 1]


def _combine_body(p_ref, o_ref):
    o_ref[...] = p_ref[0, :N] + p_ref[1, :N]


_sc_mesh = plsc.VectorSubcoreMesh(
    core_axis_name="c", subcore_axis_name="s", num_cores=NC, num_subcores=NS
)


@functools.partial(
    pl.kernel,
    out_type=jax.ShapeDtypeStruct((NC * N_PAD, D), jnp.float32),
    mesh=_sc_mesh,
    scratch_types=[
        [
            dict(
                elg=pltpu.VMEM((C,), jnp.float32),
                erg=pltpu.VMEM((C,), jnp.float32),
                a=pltpu.VMEM((C,), jnp.float32),
                rows=pltpu.VMEM((C, D), jnp.float32),
                gsem=pltpu.SemaphoreType.DMA,
                ssem=pltpu.SemaphoreType.DMA,
            )
            for _ in range(NBUF)
        ],
        [
            dict(
                src=pltpu.VMEM((C,), jnp.int32),
                dst=pltpu.VMEM((C,), jnp.int32),
                isem=pltpu.SemaphoreType.DMA,
            )
            for _ in range(NIB)
        ],
        pltpu.VMEM_SHARED((N_PAD, D), jnp.float32),
    ],
    compiler_params=pltpu.CompilerParams(needs_layout_passes=False),
)
def _sc_edges(ei_hbm, el_hbm, er_hbm, feat_hbm, out_hbm,
              rbufs, ibufs, acc):
    cid = lax.axis_index("c")
    sid = lax.axis_index("s")
    wid = sid * NC + cid
    ebase = wid * EPW

    def _fire_idx(ib, chunk):
        base = ebase + chunk * C
        pltpu.async_copy(ei_hbm.at[pl.ds(base, C)], ib["src"], ib["isem"])
        pltpu.async_copy(ei_hbm.at[pl.ds(E + base, C)], ib["dst"], ib["isem"])

    def _wait_idx(ib):
        pltpu.make_async_copy(ei_hbm.at[pl.ds(0, C)], ib["src"], ib["isem"]).wait()
        pltpu.make_async_copy(ei_hbm.at[pl.ds(0, C)], ib["dst"], ib["isem"]).wait()

    def _fire_gather(rb, ib):
        pltpu.async_copy(el_hbm.at[ib["src"]], rb["elg"], rb["gsem"])
        pltpu.async_copy(er_hbm.at[ib["dst"]], rb["erg"], rb["gsem"])
        pltpu.async_copy(feat_hbm.at[ib["src"]], rb["rows"], rb["gsem"])

    def _wait_gather_scalars(rb):
        pltpu.make_async_copy(el_hbm.at[pl.ds(0, C)], rb["elg"], rb["gsem"]).wait()
        pltpu.make_async_copy(er_hbm.at[pl.ds(0, C)], rb["erg"], rb["gsem"]).wait()

    def _wait_gather_rows(rb):
        pltpu.make_async_copy(feat_hbm.at[pl.ds(0, C)], rb["rows"], rb["gsem"]).wait()

    def _wait_scatter(rb):
        pltpu.make_async_copy(feat_hbm.at[pl.ds(0, C)], rb["rows"], rb["ssem"]).wait()

    z = rbufs[0]["rows"]

    def _zrow(r, carry):
        for g in range(D // L):
            z[r, pl.ds(g * L, L)] = jnp.zeros((L,), jnp.float32)
        return carry

    lax.fori_loop(0, RCH, _zrow, 0)
    row0 = sid * RPT
    for j in range(NRC):
        pltpu.sync_copy(z, acc.at[pl.ds(row0 + j * RCH, RCH)])
    plsc.subcore_barrier()

    for j in range(4):
        _fire_idx(ibufs[j], j)
    for j in range(2):
        _wait_idx(ibufs[j])
        _fire_gather(rbufs[j], ibufs[j])

    def _slot(i, carry):
        m8 = lax.rem(i, jnp.int32(NIB))
        for s in range(NIB):
            b = s % NBUF

            @pl.when(m8 == s)
            def _case():
                rb = rbufs[b]
                ib = ibufs[s]

                @pl.when(i + 4 < NCHUNK)
                def _pf_idx():
                    _fire_idx(ibufs[(s + 4) % NIB], i + 4)

                @pl.when(i + 2 < NCHUNK)
                def _pf_rows():
                    @pl.when(i >= 2)
                    def _dr():
                        _wait_scatter(rbufs[(b + 2) % NBUF])

                    ib2 = ibufs[(s + 2) % NIB]
                    _wait_idx(ib2)
                    _fire_gather(rbufs[(b + 2) % NBUF], ib2)

                _wait_gather_scalars(rb)
                for g in range(C // L):
                    e = (rb["elg"][pl.ds(g * L, L)]
                         + rb["erg"][pl.ds(g * L, L)])
                    rb["a"][pl.ds(g * L, L)] = jnp.where(e > 0, e, 0.2 * e)

                _wait_gather_rows(rb)
                av = rb["a"]
                rv = rb["rows"]

                @plsc.parallel_loop(0, C, unroll=8)
                def _row(r):
                    bc = plsc.load_gather(av, [jnp.zeros((L,), jnp.int32) + r])
                    for g in range(D // L):
                        sl = (r, pl.ds(g * L, L))
                        rv[sl] = rv[sl] * bc

                pltpu.async_copy(rv, acc.at[ib["dst"]], rb["ssem"], add=True)

        return carry

    lax.fori_loop(0, NCHUNK, _slot, 0)
    for b in range(NBUF):
        _wait_scatter(rbufs[b])

    plsc.subcore_barrier()
    st = rbufs[0]["rows"]
    for j in range(NRC):
        r0 = sid * RPT + j * RCH
        pltpu.sync_copy(acc.at[pl.ds(r0, RCH)], st)
        pltpu.sync_copy(st, out_hbm.at[pl.ds(cid * N_PAD + r0, RCH)])


def kernel(feat, edge_index, wl, bl, wr, br):
    w2 = jnp.concatenate([wl, wr], axis=1)
    b2 = jnp.concatenate([bl, br]).reshape(1, 2)
    el, er = pl.pallas_call(
        _elr_body,
        out_shape=(
            jax.ShapeDtypeStruct((N,), jnp.float32),
            jax.ShapeDtypeStruct((N,), jnp.float32),
        ),
    )(feat, w2, b2)
    ei = edge_index.astype(jnp.int32).reshape(2 * E)
    parts = _sc_edges(ei, el, er, feat)
    out = pl.pallas_call(
        _combine_body,
        out_shape=jax.ShapeDtypeStruct((N, D), jnp.float32),
    )(parts.reshape(NC, N_PAD, D))
    return out


# speedup vs baseline: 39.2031x; 1.0041x over previous
import functools

import jax
import jax.numpy as jnp
from jax import lax
from jax.experimental import pallas as pl
from jax.experimental.pallas import tpu as pltpu
from jax.experimental.pallas import tpu_sc as plsc

N = 10000
E = 320000
D = 128
L = 16
NC = 2
NS = 16
NW = NC * NS
EPW = E // NW
C = 80
NCHUNK = EPW // C
N_PAD = 10240
RPT = N_PAD // NS
RCH = C
NRC = RPT // RCH
NBUF = 4
NIB = 8


def _elr_body(feat_ref, w_ref, b_ref, el_ref, er_ref):
    elr = (
        jnp.dot(feat_ref[...], w_ref[...], preferred_element_type=jnp.float32)
        + b_ref[...]
    )
    el_ref[...] = elr[:, 0]
    er_ref[...] = elr[:, 1]


def _combine_body(p_ref, o_ref):
    o_ref[...] = p_ref[0, :N] + p_ref[1, :N]


_sc_mesh = plsc.VectorSubcoreMesh(
    core_axis_name="c", subcore_axis_name="s", num_cores=NC, num_subcores=NS
)


@functools.partial(
    pl.kernel,
    out_type=jax.ShapeDtypeStruct((NC * N_PAD, D), jnp.float32),
    mesh=_sc_mesh,
    scratch_types=[
        [
            dict(
                elg=pltpu.VMEM((C,), jnp.float32),
                erg=pltpu.VMEM((C,), jnp.float32),
                a=pltpu.VMEM((C,), jnp.float32),
                rows=pltpu.VMEM((C, D), jnp.float32),
                gsem=pltpu.SemaphoreType.DMA,
                ssem=pltpu.SemaphoreType.DMA,
            )
            for _ in range(NBUF)
        ],
        [
            dict(
                src=pltpu.VMEM((C,), jnp.int32),
                dst=pltpu.VMEM((C,), jnp.int32),
                isem=pltpu.SemaphoreType.DMA,
            )
            for _ in range(NIB)
        ],
        pltpu.VMEM_SHARED((N_PAD, D), jnp.float32),
    ],
    compiler_params=pltpu.CompilerParams(needs_layout_passes=False),
)
def _sc_edges(ei_hbm, el_hbm, er_hbm, feat_hbm, out_hbm,
              rbufs, ibufs, acc):
    cid = lax.axis_index("c")
    sid = lax.axis_index("s")
    wid = sid * NC + cid
    ebase = wid * EPW

    def _fire_idx(ib, chunk):
        base = ebase + chunk * C
        pltpu.async_copy(ei_hbm.at[pl.ds(base, C)], ib["src"], ib["isem"])
        pltpu.async_copy(ei_hbm.at[pl.ds(E + base, C)], ib["dst"], ib["isem"])

    def _wait_idx(ib):
        pltpu.make_async_copy(ei_hbm.at[pl.ds(0, C)], ib["src"], ib["isem"]).wait()
        pltpu.make_async_copy(ei_hbm.at[pl.ds(0, C)], ib["dst"], ib["isem"]).wait()

    def _fire_gather(rb, ib):
        pltpu.async_copy(el_hbm.at[ib["src"]], rb["elg"], rb["gsem"])
        pltpu.async_copy(er_hbm.at[ib["dst"]], rb["erg"], rb["gsem"])
        pltpu.async_copy(feat_hbm.at[ib["src"]], rb["rows"], rb["gsem"])

    def _wait_gather_scalars(rb):
        pltpu.make_async_copy(el_hbm.at[pl.ds(0, C)], rb["elg"], rb["gsem"]).wait()
        pltpu.make_async_copy(er_hbm.at[pl.ds(0, C)], rb["erg"], rb["gsem"]).wait()

    def _wait_gather_rows(rb):
        pltpu.make_async_copy(feat_hbm.at[pl.ds(0, C)], rb["rows"], rb["gsem"]).wait()

    def _wait_scatter(rb):
        pltpu.make_async_copy(out_hbm.at[pl.ds(0, C)], rb["rows"], rb["ssem"]).wait()

    z = rbufs[0]["rows"]

    def _zrow(r, carry):
        for g in range(D // L):
            z[r, pl.ds(g * L, L)] = jnp.zeros((L,), jnp.float32)
        return carry

    lax.fori_loop(0, RCH, _zrow, 0)
    row0 = sid * RPT
    for j in range(NRC):
        pltpu.sync_copy(z, acc.at[pl.ds(row0 + j * RCH, RCH)])
    plsc.subcore_barrier()

    for j in range(4):
        _fire_idx(ibufs[j], j)
    for j in range(2):
        _wait_idx(ibufs[j])
        _fire_gather(rbufs[j], ibufs[j])

    def _slot(i, carry):
        m8 = lax.rem(i, jnp.int32(NIB))
        for s in range(NIB):
            b = s % NBUF

            @pl.when(m8 == s)
            def _case():
                rb = rbufs[b]
                ib = ibufs[s]

                @pl.when(i + 4 < NCHUNK)
                def _pf_idx():
                    _fire_idx(ibufs[(s + 4) % NIB], i + 4)

                @pl.when(i + 2 < NCHUNK)
                def _pf_rows():
                    @pl.when(i >= 2)
                    def _dr():
                        _wait_scatter(rbufs[(b + 2) % NBUF])

                    ib2 = ibufs[(s + 2) % NIB]
                    _wait_idx(ib2)
                    _fire_gather(rbufs[(b + 2) % NBUF], ib2)

                _wait_gather_scalars(rb)
                for g in range(C // L):
                    e = (rb["elg"][pl.ds(g * L, L)]
                         + rb["erg"][pl.ds(g * L, L)])
                    rb["a"][pl.ds(g * L, L)] = jnp.where(e > 0, e, 0.2 * e)

                _wait_gather_rows(rb)
                av = rb["a"]
                rv = rb["rows"]

                @plsc.parallel_loop(0, C, unroll=8)
                def _row(r):
                    bc = plsc.load_gather(av, [jnp.zeros((L,), jnp.int32) + r])
                    for g in range(D // L):
                        sl = (r, pl.ds(g * L, L))
                        rv[sl] = rv[sl] * bc

                pltpu.async_copy(rv, acc.at[ib["dst"]], rb["ssem"], add=True)

        return carry

    lax.fori_loop(0, NCHUNK, _slot, 0)
    for b in range(NBUF):
        _wait_scatter(rbufs[b])

    plsc.subcore_barrier()
    st = rbufs[0]["rows"]
    for j in range(NRC):
        r0 = sid * RPT + j * RCH
        pltpu.sync_copy(acc.at[pl.ds(r0, RCH)], st)
        pltpu.sync_copy(st, out_hbm.at[pl.ds(cid * N_PAD + r0, RCH)])


def kernel(feat, edge_index, wl, bl, wr, br):
    w2 = jnp.concatenate([wl, wr], axis=1)
    b2 = jnp.concatenate([bl, br]).reshape(1, 2)
    el, er = pl.pallas_call(
        _elr_body,
        out_shape=(
            jax.ShapeDtypeStruct((N,), jnp.float32),
            jax.ShapeDtypeStruct((N,), jnp.float32),
        ),
    )(feat, w2, b2)
    ei = edge_index.astype(jnp.int32).reshape(2 * E)
    parts = _sc_edges(ei, el, er, feat)
    out = pl.pallas_call(
        _combine_body,
        out_shape=jax.ShapeDtypeStruct((N, D), jnp.float32),
    )(parts.reshape(NC, N_PAD, D))
    return out
